# Initial kernel scaffold; baseline (speedup 1.0000x reference)
import jax
import jax.numpy as jnp
from jax.experimental import pallas as pl


def kernel(query_embeds, doc_embeds, soft_labels, num_docs_per_sample):
    raise NotImplementedError("write your pallas kernel here")


# speedup vs baseline: 1.9250x; 1.9250x over previous
import jax
import jax.numpy as jnp
from jax.experimental import pallas as pl
from jax.experimental.pallas import tpu as pltpu

B = 16
D = 768
MAXD = 512
NDOCS = B * MAXD
W = MAXD + 8
INV_T = 50.0


def _body(nd_smem, q_ref, labels_ref, docs_any, out_ref, buf, simbuf, sem):
    b = pl.program_id(0)
    off = jax.lax.fori_loop(
        0, b, lambda j, a: a + nd_smem[j], jnp.int32(0), unroll=True
    )
    nd_b = nd_smem[b]
    astart = jnp.minimum(8 * (off // 8), NDOCS - W)
    shift = off - astart

    cp = pltpu.make_async_copy(
        docs_any.at[pl.ds(pl.multiple_of(astart, 8), W)], buf, sem
    )
    cp.start()
    cp.wait()

    onehot = (jax.lax.broadcasted_iota(jnp.int32, (1, B), 1) == b).astype(
        jnp.float32
    )
    q_row = jax.lax.dot_general(
        onehot, q_ref[...], (((1,), (0,)), ((), ())),
        preferred_element_type=jnp.float32,
        precision=jax.lax.Precision.HIGHEST,
    )
    sim = jax.lax.dot_general(
        buf[...], q_row, (((1,), (1,)), ((), ())),
        preferred_element_type=jnp.float32,
        precision=jax.lax.Precision.HIGHEST,
    )
    simbuf[0:W, 0:1] = sim * INV_T
    simbuf[W : 2 * MAXD, 0:1] = jnp.full((2 * MAXD - W, 1), -jnp.inf, jnp.float32)
    simw = simbuf[pl.ds(shift, MAXD), 0:1]

    pos = jax.lax.broadcasted_iota(jnp.int32, (MAXD, 1), 0)
    mask = pos < nd_b
    sims = jnp.where(mask, simw, -jnp.inf)
    mx = jnp.max(sims, axis=0, keepdims=True)
    mxs = jnp.where(nd_b > 0, mx, 0.0)
    ex = jnp.where(mask, jnp.exp(sims - mxs), 0.0)
    sexp = jnp.sum(ex, axis=0, keepdims=True)
    logz = jnp.log(sexp)

    labels_col = jax.lax.dot_general(
        labels_ref[...], onehot, (((1,), (1,)), ((), ())),
        preferred_element_type=jnp.float32,
        precision=jax.lax.Precision.HIGHEST,
    )
    pt = jnp.where(mask, labels_col, 0.0)
    s = jnp.sum(pt, axis=0, keepdims=True) + 1e-9
    pt = pt / s
    logpt = jnp.log(jnp.where(pt > 0, pt, 1.0))
    logsm = sims - mxs - logz
    terms = jnp.where(mask, pt * logpt - pt * logsm, 0.0)
    loss_b = jnp.sum(terms)

    @pl.when(b == 0)
    def _():
        out_ref[0, 0] = 0.0

    out_ref[0, 0] += loss_b * (1.0 / B)


def kernel(query_embeds, doc_embeds, soft_labels, num_docs_per_sample):
    nd = num_docs_per_sample.astype(jnp.int32)
    labels_t = soft_labels.T
    out = pl.pallas_call(
        _body,
        grid=(B,),
        in_specs=[
            pl.BlockSpec(memory_space=pltpu.SMEM),
            pl.BlockSpec((B, D), lambda b: (0, 0)),
            pl.BlockSpec((MAXD, B), lambda b: (0, 0)),
            pl.BlockSpec(memory_space=pl.ANY),
        ],
        out_specs=pl.BlockSpec(memory_space=pltpu.SMEM),
        out_shape=jax.ShapeDtypeStruct((1, 1), jnp.float32),
        scratch_shapes=[
            pltpu.VMEM((W, D), jnp.float32),
            pltpu.VMEM((2 * MAXD, 128), jnp.float32),
            pltpu.SemaphoreType.DMA,
        ],
    )(nd, query_embeds, labels_t, doc_embeds)
    return out[0, 0]
